# baseline (device time: 40800 ns/iter reference)
import jax
import jax.numpy as jnp
from jax import lax
from jax.experimental import pallas as pl
from jax.experimental.pallas import tpu as pltpu


def kernel(x, Win0, Wout0, Win1, Wout1, Win2, Wout2):
    m, d_loc = x.shape
    _, h_loc = Win0.shape
    bf16 = jnp.bfloat16

    def body(x_ref, win0_ref, wout0_ref, win1_ref, wout1_ref, win2_ref,
             wout2_ref, out_ref, win_buf, wout_buf, hsend, hrecv, gsend,
             grecv, send_sems, recv_sems, win_sem, wout_sem):
        my_x = lax.axis_index("x")
        my_y = lax.axis_index("y")
        x_partner = (1 - my_x, my_y)
        y_partner = (my_x, 1 - my_y)

        wins = (win0_ref, win1_ref, win2_ref)
        wouts = (wout0_ref, wout1_ref, wout2_ref)

        def copy_win(l):
            return pltpu.make_async_copy(wins[l], win_buf, win_sem)

        def copy_wout(l):
            return pltpu.make_async_copy(wouts[l], wout_buf, wout_sem)

        copy_win(0).start()
        copy_wout(0).start()

        barrier = pltpu.get_barrier_semaphore()
        for nbr in (x_partner, y_partner):
            pl.semaphore_signal(barrier, inc=1, device_id=nbr,
                                device_id_type=pl.DeviceIdType.MESH)
        pl.semaphore_wait(barrier, 2)

        h_rdmas = []
        g_rdmas = []
        acc = x_ref[...].astype(bf16)
        copy_win(0).wait()
        win_c = win_buf[...].astype(bf16)
        for l in range(3):
            hp = jnp.dot(acc, win_c, preferred_element_type=jnp.float32)
            if l < 2:
                copy_win(l + 1).start()
            hsend[l] = hp.astype(bf16)
            rdma = pltpu.make_async_remote_copy(
                src_ref=hsend.at[l], dst_ref=hrecv.at[l],
                send_sem=send_sems.at[2 * l], recv_sem=recv_sems.at[2 * l],
                device_id=y_partner, device_id_type=pl.DeviceIdType.MESH)
            rdma.start()
            h_rdmas.append(rdma)
            copy_wout(l).wait()
            wout_c = wout_buf[...].astype(bf16)
            rdma.wait_recv()
            h = jnp.maximum(hp + hrecv[l].astype(jnp.float32), 0.0)

            gp = jnp.dot(h.astype(bf16), wout_c,
                         preferred_element_type=jnp.float32)
            if l < 2:
                copy_wout(l + 1).start()
            gsend[l] = gp.astype(bf16)
            rdma = pltpu.make_async_remote_copy(
                src_ref=gsend.at[l], dst_ref=grecv.at[l],
                send_sem=send_sems.at[2 * l + 1],
                recv_sem=recv_sems.at[2 * l + 1],
                device_id=x_partner, device_id_type=pl.DeviceIdType.MESH)
            rdma.start()
            g_rdmas.append(rdma)
            if l < 2:
                copy_win(l + 1).wait()
                win_c = win_buf[...].astype(bf16)
            rdma.wait_recv()
            accf = gp + grecv[l].astype(jnp.float32)
            if l == 2:
                out_ref[...] = accf
            else:
                acc = accf.astype(bf16)

        for rdma in h_rdmas + g_rdmas:
            rdma.wait_send()

    return pl.pallas_call(
        body,
        out_shape=jax.ShapeDtypeStruct((m, d_loc), jnp.float32),
        in_specs=[pl.BlockSpec(memory_space=pltpu.VMEM)]
        + [pl.BlockSpec(memory_space=pltpu.MemorySpace.HBM)] * 6,
        out_specs=pl.BlockSpec(memory_space=pltpu.VMEM),
        scratch_shapes=[
            pltpu.VMEM((d_loc, h_loc), jnp.float32),
            pltpu.VMEM((h_loc, d_loc), jnp.float32),
            pltpu.VMEM((3, m, h_loc), bf16),
            pltpu.VMEM((3, m, h_loc), bf16),
            pltpu.VMEM((3, m, d_loc), bf16),
            pltpu.VMEM((3, m, d_loc), bf16),
            pltpu.SemaphoreType.DMA((6,)),
            pltpu.SemaphoreType.DMA((6,)),
            pltpu.SemaphoreType.DMA,
            pltpu.SemaphoreType.DMA,
        ],
        compiler_params=pltpu.CompilerParams(collective_id=0),
    )(x, Win0, Wout0, Win1, Wout1, Win2, Wout2)


# device time: 33001 ns/iter; 1.2363x vs baseline; 1.2363x over previous
import jax
import jax.numpy as jnp
from jax import lax
from jax.experimental import pallas as pl
from jax.experimental.pallas import tpu as pltpu


def kernel(x, Win0, Wout0, Win1, Wout1, Win2, Wout2):
    m, d_loc = x.shape
    _, h_loc = Win0.shape
    bf16 = jnp.bfloat16

    def body(x_ref, win0_ref, wout0_ref, win1_ref, wout1_ref, win2_ref,
             wout2_ref, out_ref, win_bufs, wout_bufs, win_bf, wout_bf,
             hsend, hrecv, gsend, grecv, send_sems, recv_sems, win_sems,
             wout_sems, win0_sems, out_vmem, out_sems):
        my_x = lax.axis_index("x")
        my_y = lax.axis_index("y")
        x_partner = (1 - my_x, my_y)
        y_partner = (my_x, 1 - my_y)

        wins = (win0_ref, win1_ref, win2_ref)
        wouts = (wout0_ref, wout1_ref, wout2_ref)

        def copy_win(l):
            return pltpu.make_async_copy(
                wins[l], win_bufs.at[l % 2], win_sems.at[l % 2])

        def copy_wout(l):
            return pltpu.make_async_copy(
                wouts[l], wout_bufs.at[l % 2], wout_sems.at[l % 2])

        half = h_loc // 2

        def copy_win0_half(k):
            return pltpu.make_async_copy(
                win0_ref.at[:, pl.ds(k * half, half)],
                win_bufs.at[0, :, pl.ds(k * half, half)],
                win0_sems.at[k])

        copy_win0_half(0).start()
        copy_win0_half(1).start()
        copy_wout(0).start()
        copy_win(1).start()
        copy_wout(1).start()

        barrier = pltpu.get_barrier_semaphore()
        for nbr in (x_partner, y_partner):
            pl.semaphore_signal(barrier, inc=1, device_id=nbr,
                                device_id_type=pl.DeviceIdType.MESH)
        pl.semaphore_wait(barrier, 2)

        NC = 4
        hw = h_loc // NC
        gw = d_loc // NC

        all_rdmas = []
        acc = x_ref[...].astype(bf16)
        accf = None
        for l in range(3):
            s, s1 = l % 2, (l + 1) % 2

            hps = []
            y_rdmas = []
            for c in range(NC):
                if l == 0:
                    if c * hw % half == 0:
                        k = (c * hw) // half
                        copy_win0_half(k).wait()
                        win_bf[0, :, k * half:(k + 1) * half] = (
                            win_bufs[0, :, k * half:(k + 1) * half].astype(bf16))
                    hp_c = jnp.dot(acc, win_bf[s, :, c * hw:(c + 1) * hw],
                                   preferred_element_type=jnp.float32)
                else:
                    hp_c = hp_full[:, c * hw:(c + 1) * hw]
                hsend[l, :, c * hw:(c + 1) * hw] = hp_c.astype(bf16)
                rdma = pltpu.make_async_remote_copy(
                    src_ref=hsend.at[l, :, pl.ds(c * hw, hw)],
                    dst_ref=hrecv.at[l, :, pl.ds(c * hw, hw)],
                    send_sem=send_sems.at[2 * l, c],
                    recv_sem=recv_sems.at[2 * l, c],
                    device_id=y_partner, device_id_type=pl.DeviceIdType.MESH)
                rdma.start()
                hps.append(hp_c)
                y_rdmas.append(rdma)
            all_rdmas += y_rdmas
            if l == 0:
                copy_win(2).start()
            copy_wout(l).wait()
            wout_bf[s] = wout_bufs[s].astype(bf16)
            if l == 0:
                copy_wout(2).start()

            gp = None
            for c in range(NC):
                y_rdmas[c].wait_recv()
                h_c = jnp.maximum(
                    hps[c] + hrecv[l, :, c * hw:(c + 1) * hw].astype(jnp.float32),
                    0.0).astype(bf16)
                t = jnp.dot(h_c, wout_bf[s, c * hw:(c + 1) * hw, :],
                            preferred_element_type=jnp.float32)
                gp = t if gp is None else gp + t

            gps = []
            x_rdmas = []
            for c in range(NC):
                gp_c = gp[:, c * gw:(c + 1) * gw]
                gsend[l, :, c * gw:(c + 1) * gw] = gp_c.astype(bf16)
                rdma = pltpu.make_async_remote_copy(
                    src_ref=gsend.at[l, :, pl.ds(c * gw, gw)],
                    dst_ref=grecv.at[l, :, pl.ds(c * gw, gw)],
                    send_sem=send_sems.at[2 * l + 1, c],
                    recv_sem=recv_sems.at[2 * l + 1, c],
                    device_id=x_partner, device_id_type=pl.DeviceIdType.MESH)
                rdma.start()
                gps.append(gp_c)
                x_rdmas.append(rdma)
            all_rdmas += x_rdmas
            if l < 2:
                copy_win(l + 1).wait()
                win_bf[s1] = win_bufs[s1].astype(bf16)
            hp_full = None
            for c in range(NC):
                x_rdmas[c].wait_recv()
                accf_c = gps[c] + grecv[l, :, c * gw:(c + 1) * gw].astype(
                    jnp.float32)
                if l == 2:
                    out_vmem[:, c * gw:(c + 1) * gw] = accf_c
                    pltpu.make_async_copy(
                        out_vmem.at[:, pl.ds(c * gw, gw)],
                        out_ref.at[:, pl.ds(c * gw, gw)],
                        out_sems.at[c]).start()
                else:
                    t = jnp.dot(accf_c.astype(bf16),
                                win_bf[s1, c * gw:(c + 1) * gw, :],
                                preferred_element_type=jnp.float32)
                    hp_full = t if hp_full is None else hp_full + t
        for c in range(NC):
            pltpu.make_async_copy(
                out_vmem.at[:, pl.ds(c * gw, gw)],
                out_ref.at[:, pl.ds(c * gw, gw)],
                out_sems.at[c]).wait()
        for rdma in all_rdmas:
            rdma.wait_send()

    return pl.pallas_call(
        body,
        out_shape=jax.ShapeDtypeStruct((m, d_loc), jnp.float32),
        in_specs=[pl.BlockSpec(memory_space=pltpu.VMEM)]
        + [pl.BlockSpec(memory_space=pltpu.MemorySpace.HBM)] * 6,
        out_specs=pl.BlockSpec(memory_space=pltpu.MemorySpace.HBM),
        scratch_shapes=[
            pltpu.VMEM((2, d_loc, h_loc), jnp.float32),
            pltpu.VMEM((2, h_loc, d_loc), jnp.float32),
            pltpu.VMEM((2, d_loc, h_loc), bf16),
            pltpu.VMEM((2, h_loc, d_loc), bf16),
            pltpu.VMEM((3, m, h_loc), bf16),
            pltpu.VMEM((3, m, h_loc), bf16),
            pltpu.VMEM((3, m, d_loc), bf16),
            pltpu.VMEM((3, m, d_loc), bf16),
            pltpu.SemaphoreType.DMA((6, 4)),
            pltpu.SemaphoreType.DMA((6, 4)),
            pltpu.SemaphoreType.DMA((2,)),
            pltpu.SemaphoreType.DMA((2,)),
            pltpu.SemaphoreType.DMA((2,)),
            pltpu.VMEM((m, d_loc), jnp.float32),
            pltpu.SemaphoreType.DMA((4,)),
        ],
        compiler_params=pltpu.CompilerParams(
            collective_id=0,
            vmem_limit_bytes=60 * 1024 * 1024,
        ),
    )(x, Win0, Wout0, Win1, Wout1, Win2, Wout2)
